# baseline (speedup 1.0000x reference)
import functools

import jax
import jax.numpy as jnp
from jax import lax
from jax.experimental import pallas as pl
from jax.experimental.pallas import tpu as pltpu
from jax.experimental.pallas import tpu_sc as plsc

B = 16384
L = 200
NC = 2
NS = 16
NW = NC * NS
ROWS_PER_W = B // NW
CHUNK = 128
NCHUNK = ROWS_PER_W // CHUNK

_mesh = plsc.VectorSubcoreMesh(core_axis_name="c", subcore_axis_name="s")


@functools.partial(
    pl.kernel,
    out_type=jax.ShapeDtypeStruct((B,), jnp.float32),
    mesh=_mesh,
    compiler_params=pltpu.CompilerParams(needs_layout_passes=False),
    scratch_types=[
        pltpu.VMEM((CHUNK, L), jnp.int32),
        pltpu.VMEM((CHUNK, L), jnp.int32),
        pltpu.VMEM((ROWS_PER_W,), jnp.float32),
        pltpu.VMEM((144,), jnp.float32),
        pltpu.VMEM((CHUNK, 16), jnp.float32),
        pltpu.SemaphoreType.DMA,
        pltpu.SemaphoreType.DMA,
    ],
)
def _sc_kernel(ids_hbm, par_hbm, out_hbm, ids_v0, ids_v1, out_v, par_v, cnt_v,
               sem0, sem1):
    wid = lax.axis_index("s") * NC + lax.axis_index("c")
    base_row = wid * ROWS_PER_W

    bufs = (ids_v0, ids_v1)
    sems = (sem0, sem1)

    cps = [pltpu.async_copy(
        ids_hbm.at[pl.ds(base_row, CHUNK), :], ids_v0, sem0)]
    pltpu.sync_copy(par_hbm, par_v)

    embT = [par_v[pl.ds(d * 16, 16)] for d in range(8)]
    wbv = par_v[pl.ds(128, 16)]
    b_vec = jnp.full((16,), 1.0, jnp.float32) * wbv[8]

    lane = lax.iota(jnp.int32, 16)
    zero_f = jnp.zeros((16,), jnp.float32)
    ones_f = jnp.full((16,), 1.0, jnp.float32)

    def round_bf16(x):
        u = plsc.bitcast(x, jnp.uint32)
        r = (u + jnp.uint32(0x7FFF) + ((u >> jnp.uint32(16)) & jnp.uint32(1))
             ) & jnp.uint32(0xFFFF0000)
        return plsc.bitcast(r, jnp.float32)

    for c in range(NCHUNK):
        if c + 1 < NCHUNK:
            cps.append(pltpu.async_copy(
                ids_hbm.at[pl.ds(base_row + (c + 1) * CHUNK, CHUNK), :],
                bufs[(c + 1) % 2], sems[(c + 1) % 2]))
        cps[c].wait()
        ids_v = bufs[c % 2]

        @plsc.parallel_loop(0, CHUNK // 16, unroll=2)
        def _loop(sg):
            rowv = sg * 16 + lane

            for k in range(16):
                plsc.store_scatter(cnt_v, [rowv, jnp.full((16,), k, jnp.int32)],
                                   zero_f)

            def hist_body(i, colv):
                for _ in range(8):
                    idv = plsc.load_gather(ids_v, [rowv, colv])
                    plsc.addupdate_scatter(cnt_v, [rowv, idv], ones_f)
                    colv = colv + 1
                return colv
            lax.fori_loop(0, L // 8, hist_body, jnp.zeros((16,), jnp.int32))

            acc = [zero_f] * 8
            for k in range(16):
                ck = plsc.load_gather(
                    cnt_v, [rowv, jnp.full((16,), k, jnp.int32)])
                for d in range(8):
                    acc[d] = acc[d] + ck * embT[d][k]

            out16 = b_vec
            for d in range(8):
                out16 = out16 + round_bf16(acc[d]) * wbv[d]
            out_v[pl.ds(c * CHUNK + sg * 16, 16)] = out16

    pltpu.sync_copy(out_v, out_hbm.at[pl.ds(base_row, ROWS_PER_W)])


def _round_bf16_host(x):
    u = lax.bitcast_convert_type(x.astype(jnp.float32), jnp.uint32)
    r = (u + 0x7FFF + ((u >> 16) & 1)) & jnp.uint32(0xFFFF0000)
    return lax.bitcast_convert_type(r, jnp.float32)


def kernel(input_ids, emb_table, W, b):
    params = jnp.concatenate([
        emb_table.T.reshape(-1).astype(jnp.float32),
        _round_bf16_host(W.reshape(-1)),
        b.reshape(-1).astype(jnp.float32),
        jnp.zeros((7,), jnp.float32),
    ])
    return _sc_kernel(input_ids, params).reshape(B, 1)
